# baseline (speedup 1.0000x reference)
import functools

import jax
import jax.numpy as jnp
from jax import lax
from jax.experimental import pallas as pl
from jax.experimental.pallas import tpu as pltpu
from jax.experimental.pallas import tpu_sc as plsc

_NC, _NS, _LANES = 2, 16, 16
_NW = _NC * _NS
_CHUNK = 128


def _make_sc_call(N, D, n_rows):
    zero_row = n_rows - 1
    per_w = N // _NW
    n_chunks = per_w // _CHUNK

    mesh = plsc.VectorSubcoreMesh(
        core_axis_name="c", subcore_axis_name="s",
        num_cores=_NC, num_subcores=_NS)

    @functools.partial(
        pl.kernel,
        out_type=jax.ShapeDtypeStruct((N, D), jnp.float32),
        mesh=mesh,
        scratch_types=[
            pltpu.VMEM((_CHUNK,), jnp.int32),
            pltpu.VMEM((_CHUNK,), jnp.int32),
            pltpu.VMEM((_CHUNK,), jnp.int32),
            pltpu.VMEM((_CHUNK, 128), jnp.float32),
            pltpu.VMEM((_CHUNK, D), jnp.float32),
            pltpu.SemaphoreType.DMA,
            pltpu.SemaphoreType.DMA,
        ],
    )
    def sc_call(x_hbm, foc_hbm, msk_hbm, pe_hbm, out_hbm,
                foc_v, msk_v, idx_v, rows_v, x_v, sem_x, sem_g):
        wid = lax.axis_index("s") * _NC + lax.axis_index("c")
        base_w = wid * per_w

        def chunk_body(i, carry):
            base = base_w + i * _CHUNK
            pltpu.sync_copy(foc_hbm.at[pl.ds(base, _CHUNK)], foc_v)
            pltpu.sync_copy(msk_hbm.at[pl.ds(base, _CHUNK)], msk_v)
            cp_x = pltpu.async_copy(x_hbm.at[pl.ds(base, _CHUNK), :], x_v,
                                    sem_x)

            def idx_body(j, c):
                sl = pl.ds(j * _LANES, _LANES)
                f = foc_v[sl]
                m = msk_v[sl]
                idx_v[sl] = jnp.where(m != 0, f, zero_row)
                return c
            lax.fori_loop(0, _CHUNK // _LANES, idx_body, 0)

            cp_g = pltpu.async_copy(pe_hbm.at[idx_v], rows_v, sem_g)
            cp_g.wait()
            cp_x.wait()

            def add_body(t, c):
                for k in range(D // _LANES):
                    sl = pl.ds(k * _LANES, _LANES)
                    plsc.addupdate(x_v.at[t, sl], rows_v[t, sl])
                return c
            lax.fori_loop(0, _CHUNK, add_body, 0)

            pltpu.sync_copy(x_v, out_hbm.at[pl.ds(base, _CHUNK), :])
            return carry

        lax.fori_loop(0, n_chunks, chunk_body, 0)

    return sc_call


def kernel(X, focuses, mask, pe):
    B, L, D = X.shape
    N = B * L
    x_flat = X.reshape(N, D)
    foc = focuses.reshape(N).astype(jnp.int32)
    msk = mask.reshape(N).astype(jnp.int32)
    n_rows = pe.shape[0] + 1
    pe_pad = jnp.zeros((n_rows, 128), pe.dtype).at[:pe.shape[0], :D].set(pe)
    out = _make_sc_call(N, D, n_rows)(x_flat, foc, msk, pe_pad)
    return out.reshape(B, L, D)
